# Initial kernel scaffold; baseline (speedup 1.0000x reference)
import jax
import jax.numpy as jnp
from jax.experimental import pallas as pl


def kernel(x, edge_index, spatial_coord, params):
    raise NotImplementedError("write your pallas kernel here")


# speedup vs baseline: 3.0640x; 3.0640x over previous
import functools

import jax
import jax.numpy as jnp
from jax import lax
from jax.experimental import pallas as pl
from jax.experimental.pallas import tpu as pltpu
from jax.experimental.pallas import tpu_sc as plsc

N = 10000
E = 160000
ET = E + N
NPAD = 10240
NW = 32
ECHUNK = 2000
NCH = ET // ECHUNK
LCAP = ECHUNK + 48

_GDN = lax.GatherDimensionNumbers(
    offset_dims=(), collapsed_slice_dims=(0,), start_index_map=(0,))


def _permute(v, idx):
    return lax.gather(v, idx[:, None], _GDN, slice_sizes=(1,),
                      mode=lax.GatherScatterMode.PROMISE_IN_BOUNDS)


def _lane_iota():
    return lax.iota(jnp.int32, 16)


def _allsum(v):
    lane = _lane_iota()
    s = v
    for k in (1, 2, 4, 8):
        s = s + _permute(s, jnp.bitwise_xor(lane, k))
    return s


def _inclusive_prefix(v):
    lane = _lane_iota()
    s = v
    for k in (1, 2, 4, 8):
        shifted = _permute(s, jnp.maximum(lane - k, 0))
        s = s + jnp.where(lane >= k, shifted, 0)
    return s


SLICE = 10752
ET_P = 16 * SLICE


def _gat_sc_kernel(C16, HALF16, G, GROUP, EP, efull, relu):
    NBAT = SLICE // G
    NGROUP = NBAT // GROUP
    NROW = NPAD // 16
    assert NBAT % GROUP == 0 and NROW % EP == 0

    mesh = plsc.VectorSubcoreMesh(core_axis_name="c", subcore_axis_name="s")

    @functools.partial(
        pl.kernel,
        mesh=mesh,
        compiler_params=pltpu.CompilerParams(use_tc_tiling_on_sc=False),
        out_type=jax.ShapeDtypeStruct((2, NPAD, HALF16 + 1, 16),
                                      jnp.float32),
        scratch_types=[
            pltpu.VMEM((GROUP, G), jnp.int32),
            pltpu.VMEM((GROUP, G), jnp.int32),
            pltpu.VMEM((G, C16, 16), jnp.float32),
            pltpu.VMEM((G, C16, 16), jnp.float32),
            pltpu.VMEM((G, HALF16 + 1, 16), jnp.float32),
            pltpu.VMEM((EP, HALF16 + 1, 16), jnp.float32),
            pltpu.VMEM((C16, 16), jnp.float32),
            pltpu.VMEM((C16, 16), jnp.float32),
            pltpu.VMEM_SHARED((NPAD, HALF16 + 1, 16), jnp.float32),
            pltpu.SemaphoreType.DMA,
        ],
    )
    def k(src_h, dst_h, xl_h, xr_h, att_h, b_h, out_h,
          sidx, didx, xlg, xrg, ctb, ebuf, attv, bv, acc_sh, sem):
        cid = lax.axis_index("c")
        sid = lax.axis_index("s")
        pltpu.sync_copy(att_h, attv)
        pltpu.sync_copy(b_h, bv)
        zf = jnp.zeros((16,), jnp.float32)

        def _zc(i, _):
            for c in range(HALF16 + 1):
                ctb[i, c, :] = zf
            return 0
        lax.fori_loop(0, G, _zc, 0)

        def _za(i, _):
            pltpu.sync_copy(ctb, acc_sh.at[pl.ds(sid * NROW + i * G, G)])
            return 0
        lax.fori_loop(0, NROW // G, _za, 0)
        plsc.subcore_barrier()

        def group_body(g, _):
            pltpu.sync_copy(src_h.at[sid, pl.ds(g * GROUP, GROUP)], sidx)
            pltpu.sync_copy(dst_h.at[sid, pl.ds(g * GROUP, GROUP)], didx)

            def batch(bi, _):
                pltpu.async_copy(xl_h.at[sidx.at[bi]], xlg, sem).wait()
                pltpu.async_copy(xr_h.at[didx.at[bi]], xrg, sem).wait()

                def edge(j, _):
                    ea = zf
                    if efull:
                        for c in range(C16):
                            sv = xlg[j, c, :] + xrg[j, c, :]
                            lk = (jnp.maximum(sv, 0.0)
                                  + 0.2 * jnp.minimum(sv, 0.0))
                            ea = ea + attv[c, :] * lk
                    else:
                        for c in range(HALF16):
                            c2 = cid * HALF16 + c
                            sv = xlg[j, c2, :] + xrg[j, c2, :]
                            lk = (jnp.maximum(sv, 0.0)
                                  + 0.2 * jnp.minimum(sv, 0.0))
                            ea = ea + attv[c2, :] * lk
                    e = _allsum(ea)
                    e = jnp.minimum(jnp.maximum(e, -60.0), 60.0)
                    ex = jnp.exp(e)
                    for c in range(HALF16):
                        c2 = cid * HALF16 + c
                        ctb[j, c, :] = ex * xlg[j, c2, :]
                    ctb[j, HALF16, :] = ex
                    return 0
                lax.fori_loop(0, G, edge, 0)
                pltpu.sync_copy(ctb, acc_sh.at[didx.at[bi]], add=True)
                return 0
            lax.fori_loop(0, GROUP, batch, 0)
            return 0
        lax.fori_loop(0, NGROUP, group_body, 0)
        plsc.subcore_barrier()

        def ep(i, _):
            r0 = sid * NROW + i * EP
            pltpu.sync_copy(acc_sh.at[pl.ds(r0, EP)], ebuf)

            def nr(r, _):
                inv = 1.0 / (ebuf[r, HALF16, :] + 1e-16)
                for c in range(HALF16):
                    c2 = cid * HALF16 + c
                    v = ebuf[r, c, :] * inv + bv[c2, :]
                    if relu:
                        v = jnp.maximum(v, 0.0)
                    ebuf[r, c, :] = v
                return 0
            lax.fori_loop(0, EP, nr, 0)
            pltpu.sync_copy(ebuf, out_h.at[cid, pl.ds(r0, EP)])
            return 0
        lax.fori_loop(0, NROW // EP, ep, 0)

    return k


def _pad_rows(x):
    return jnp.pad(x, ((0, NPAD - N), (0, 0)))


def _gat_conv(xl, xr, src_p, dst_p, att, b, relu):
    C = xl.shape[1]
    C16 = C // 16
    HALF16 = C16 // 2
    G, GROUP, EP = (32, 24, 32) if C == 256 else (64, 24, 64)
    src3 = src_p.reshape(16, SLICE // G, G)
    dst3 = dst_p.reshape(16, SLICE // G, G)
    xl3 = _pad_rows(xl).reshape(NPAD, C16, 16)
    xr3 = _pad_rows(xr).reshape(NPAD, C16, 16)
    out = _gat_sc_kernel(C16, HALF16, G, GROUP, EP, True, relu)(
        src3, dst3, xl3, xr3, att.reshape(C16, 16), b.reshape(C16, 16))
    halves = out[:, :, :HALF16, :].reshape(2, NPAD, C // 2)
    return jnp.concatenate([halves[0], halves[1]], axis=-1)[:N]


def _gat_conv_pair(xlmu, xrmu, xlls, xrls, src_p, dst_p, attmu, bmu,
                   attls, bls):
    G = 128
    src3 = src_p.reshape(16, SLICE // G, G)
    dst3 = dst_p.reshape(16, SLICE // G, G)
    xl3 = _pad_rows(jnp.concatenate([xlmu, xlls], -1)).reshape(NPAD, 2, 16)
    xr3 = _pad_rows(jnp.concatenate([xrmu, xrls], -1)).reshape(NPAD, 2, 16)
    att2 = jnp.stack([attmu, attls])
    b2 = jnp.stack([bmu, bls])
    out = _gat_sc_kernel(2, 1, G, 84, 128, False, False)(
        src3, dst3, xl3, xr3, att2, b2)
    mu = out[0, :, 0, :].reshape(NPAD, 16)[:N]
    ls = out[1, :, 0, :].reshape(NPAD, 16)[:N]
    return mu, ls


def _matmul(x, w):
    M, K = x.shape
    Co = w.shape[1]
    BM = 1000

    def body(x_ref, w_ref, o_ref):
        o_ref[...] = jnp.dot(x_ref[...], w_ref[...],
                             preferred_element_type=jnp.float32)

    return pl.pallas_call(
        body,
        grid=(M // BM,),
        in_specs=[pl.BlockSpec((BM, K), lambda i: (i, 0)),
                  pl.BlockSpec((K, Co), lambda i: (0, 0))],
        out_specs=pl.BlockSpec((BM, Co), lambda i: (i, 0)),
        out_shape=jax.ShapeDtypeStruct((M, Co), jnp.float32),
    )(x, w)


def _leaky(x):
    return jnp.maximum(x, 0.0) + 0.2 * jnp.minimum(x, 0.0)


def _decoder(mu, logstd, eps, p):
    def body(mu_r, ls_r, eps_r, w1, b1, g1, be1, w2, b2, g2, be2, z_r, d_r):
        z = eps_r[...] * jnp.exp(ls_r[...]) + mu_r[...]
        z_r[...] = z
        a = jnp.dot(z, w1[...], preferred_element_type=jnp.float32) + b1[...]
        m = jnp.mean(a, axis=0, keepdims=True)
        v = jnp.mean((a - m) ** 2, axis=0, keepdims=True)
        a = _leaky(g1[...] * (a - m) / jnp.sqrt(v + 1e-5) + be1[...])
        d = jnp.dot(a, w2[...], preferred_element_type=jnp.float32) + b2[...]
        m2 = jnp.mean(d, axis=0, keepdims=True)
        v2 = jnp.mean((d - m2) ** 2, axis=0, keepdims=True)
        d_r[...] = _leaky(g2[...] * (d - m2) / jnp.sqrt(v2 + 1e-5) + be2[...])

    args = (mu, logstd, eps,
            p["W_d1"], p["b_d1"].reshape(1, -1),
            p["g1"].reshape(1, -1), p["be1"].reshape(1, -1),
            p["W_d2"], p["b_d2"].reshape(1, -1),
            p["g2"].reshape(1, -1), p["be2"].reshape(1, -1))
    return pl.pallas_call(
        body,
        out_shape=(jax.ShapeDtypeStruct((N, 16), jnp.float32),
                   jax.ShapeDtypeStruct((N, 256), jnp.float32)),
    )(*args)


def _adj(d):
    B = 400

    def body(a_ref, b_ref, o_ref):
        o_ref[...] = jax.nn.sigmoid(
            lax.dot_general(a_ref[...], b_ref[...],
                            (((1,), (1,)), ((), ())),
                            preferred_element_type=jnp.float32))

    return pl.pallas_call(
        body,
        grid=(N // B,),
        in_specs=[pl.BlockSpec((B, 256), lambda i: (i, 0)),
                  pl.BlockSpec((N, 256), lambda i: (0, 0))],
        out_specs=pl.BlockSpec((B, N), lambda i: (i, 0)),
        out_shape=jax.ShapeDtypeStruct((N, N), jnp.float32),
    )(d, d)


def _heads(d, p):
    BM = 1000
    OUT = p["W21"].shape[1]

    def body(d_ref, w1, c1, w2, c2, w3, c3, r1, r2, r3):
        x = d_ref[...]
        r1[...] = jnp.exp(
            jnp.dot(x, w1[...], preferred_element_type=jnp.float32) + c1[...])
        s = jnp.dot(x, w2[...], preferred_element_type=jnp.float32) + c2[...]
        r2[...] = jnp.clip(jax.nn.softplus(s), 1e-4, 1e4)
        r3[...] = jax.nn.sigmoid(
            jnp.dot(x, w3[...], preferred_element_type=jnp.float32) + c3[...])

    wspec = pl.BlockSpec((256, OUT), lambda i: (0, 0))
    bspec = pl.BlockSpec((1, OUT), lambda i: (0, 0))
    ospec = pl.BlockSpec((BM, OUT), lambda i: (i, 0))
    oshape = jax.ShapeDtypeStruct((N, OUT), jnp.float32)
    return pl.pallas_call(
        body,
        grid=(N // BM,),
        in_specs=[pl.BlockSpec((BM, 256), lambda i: (i, 0)),
                  wspec, bspec, wspec, bspec, wspec, bspec],
        out_specs=(ospec, ospec, ospec),
        out_shape=(oshape, oshape, oshape),
    )(d, p["W21"], p["b21"].reshape(1, -1),
      p["W22"], p["b22"].reshape(1, -1),
      p["W23"], p["b23"].reshape(1, -1))


def kernel(x, edge_index, spatial_coord, params):
    del spatial_coord
    loop = jnp.arange(N, dtype=edge_index.dtype)
    src_p = jnp.concatenate([
        edge_index[0], loop,
        jnp.zeros((ET_P - ET,), edge_index.dtype)])
    dst_p = jnp.concatenate([
        edge_index[1], loop,
        jnp.full((ET_P - ET,), NPAD - 1, edge_index.dtype)])

    def conv(h, name, relu):
        cp = params[name]
        C = cp["Wl"].shape[1]
        xlxr = _matmul(h, jnp.concatenate([cp["Wl"], cp["Wr"]], axis=1))
        return _gat_conv(xlxr[:, :C], xlxr[:, C:], src_p, dst_p,
                         cp["att"], cp["b"], relu)

    h = conv(x, "conv1", True)
    h = conv(h, "conv2", True)

    pmu, pls = params["conv_mu"], params["conv_logstd"]
    w4 = jnp.concatenate([pmu["Wl"], pmu["Wr"], pls["Wl"], pls["Wr"]], 1)
    xx = _matmul(h, w4)
    mu, logstd = _gat_conv_pair(
        xx[:, 0:16], xx[:, 16:32], xx[:, 32:48], xx[:, 48:64],
        src_p, dst_p, pmu["att"], pmu["b"], pls["att"], pls["b"])

    eps = jax.random.normal(jax.random.key(7), (N, 16), dtype=jnp.float32)
    z, d = _decoder(mu, logstd, eps, params)
    adj_pred = _adj(d)
    r1, r2, r3 = _heads(d, params)
    return (z, r1, r2, r3, mu, logstd, adj_pred)


# speedup vs baseline: 3.4125x; 1.1137x over previous
import functools

import jax
import jax.numpy as jnp
from jax import lax
from jax.experimental import pallas as pl
from jax.experimental.pallas import tpu as pltpu
from jax.experimental.pallas import tpu_sc as plsc

N = 10000
E = 160000
ET = E + N
NPAD = 10240
NW = 32
ECHUNK = 2000
NCH = ET // ECHUNK
LCAP = ECHUNK + 48

_GDN = lax.GatherDimensionNumbers(
    offset_dims=(), collapsed_slice_dims=(0,), start_index_map=(0,))


def _permute(v, idx):
    return lax.gather(v, idx[:, None], _GDN, slice_sizes=(1,),
                      mode=lax.GatherScatterMode.PROMISE_IN_BOUNDS)


def _lane_iota():
    return lax.iota(jnp.int32, 16)


def _allsum(v):
    lane = _lane_iota()
    s = v
    for k in (1, 2, 4, 8):
        s = s + _permute(s, jnp.bitwise_xor(lane, k))
    return s


def _inclusive_prefix(v):
    lane = _lane_iota()
    s = v
    for k in (1, 2, 4, 8):
        shifted = _permute(s, jnp.maximum(lane - k, 0))
        s = s + jnp.where(lane >= k, shifted, 0)
    return s


SLICE = 10752
ET_P = 16 * SLICE


def _gat_sc_kernel(C16, HALF16, G, GROUP, EP, efull, relu):
    NBAT = SLICE // G
    NGROUP = NBAT // GROUP
    NROW = NPAD // 16
    assert NBAT % GROUP == 0 and NROW % EP == 0

    mesh = plsc.VectorSubcoreMesh(core_axis_name="c", subcore_axis_name="s")

    @functools.partial(
        pl.kernel,
        mesh=mesh,
        compiler_params=pltpu.CompilerParams(use_tc_tiling_on_sc=False),
        out_type=jax.ShapeDtypeStruct((2, NPAD, HALF16 + 1, 16),
                                      jnp.float32),
        scratch_types=[
            pltpu.VMEM((GROUP, G), jnp.int32),
            pltpu.VMEM((GROUP, G), jnp.int32),
            pltpu.VMEM((G, C16, 16), jnp.float32),
            pltpu.VMEM((G, C16, 16), jnp.float32),
            pltpu.VMEM((G, HALF16 + 1, 16), jnp.float32),
            pltpu.VMEM((EP, HALF16 + 1, 16), jnp.float32),
            pltpu.VMEM((C16, 16), jnp.float32),
            pltpu.VMEM((C16, 16), jnp.float32),
            pltpu.VMEM_SHARED((NPAD, HALF16 + 1, 16), jnp.float32),
            pltpu.SemaphoreType.DMA,
        ],
    )
    def k(src_h, dst_h, xl_h, xr_h, att_h, b_h, out_h,
          sidx, didx, xlg, xrg, ctb, ebuf, attv, bv, acc_sh, sem):
        cid = lax.axis_index("c")
        sid = lax.axis_index("s")
        pltpu.sync_copy(att_h, attv)
        pltpu.sync_copy(b_h, bv)
        zf = jnp.zeros((16,), jnp.float32)

        def _zc(i, _):
            for c in range(HALF16 + 1):
                ctb[i, c, :] = zf
            return 0
        lax.fori_loop(0, G, _zc, 0)

        def _za(i, _):
            pltpu.sync_copy(ctb, acc_sh.at[pl.ds(sid * NROW + i * G, G)])
            return 0
        lax.fori_loop(0, NROW // G, _za, 0)
        plsc.subcore_barrier()

        def group_body(g, _):
            pltpu.sync_copy(src_h.at[sid, pl.ds(g * GROUP, GROUP)], sidx)
            pltpu.sync_copy(dst_h.at[sid, pl.ds(g * GROUP, GROUP)], didx)

            def batch(bi, _):
                cp1 = pltpu.async_copy(xl_h.at[sidx.at[bi]], xlg, sem)
                cp2 = pltpu.async_copy(xr_h.at[didx.at[bi]], xrg, sem)
                cp1.wait()
                cp2.wait()

                def edge(j, _):
                    ea = zf
                    if efull:
                        for c in range(C16):
                            sv = xlg[j, c, :] + xrg[j, c, :]
                            lk = (jnp.maximum(sv, 0.0)
                                  + 0.2 * jnp.minimum(sv, 0.0))
                            ea = ea + attv[c, :] * lk
                    else:
                        for c in range(HALF16):
                            c2 = cid * HALF16 + c
                            sv = xlg[j, c2, :] + xrg[j, c2, :]
                            lk = (jnp.maximum(sv, 0.0)
                                  + 0.2 * jnp.minimum(sv, 0.0))
                            ea = ea + attv[c2, :] * lk
                    e = _allsum(ea)
                    e = jnp.minimum(jnp.maximum(e, -60.0), 60.0)
                    ex = jnp.exp(e)
                    for c in range(HALF16):
                        c2 = cid * HALF16 + c
                        ctb[j, c, :] = ex * xlg[j, c2, :]
                    ctb[j, HALF16, :] = ex
                    return 0
                lax.fori_loop(0, G, edge, 0)
                pltpu.sync_copy(ctb, acc_sh.at[didx.at[bi]], add=True)
                return 0
            lax.fori_loop(0, GROUP, batch, 0)
            return 0
        lax.fori_loop(0, NGROUP, group_body, 0)
        plsc.subcore_barrier()

        def ep(i, _):
            r0 = sid * NROW + i * EP
            pltpu.sync_copy(acc_sh.at[pl.ds(r0, EP)], ebuf)

            def nr(r, _):
                inv = 1.0 / (ebuf[r, HALF16, :] + 1e-16)
                for c in range(HALF16):
                    c2 = cid * HALF16 + c
                    v = ebuf[r, c, :] * inv + bv[c2, :]
                    if relu:
                        v = jnp.maximum(v, 0.0)
                    ebuf[r, c, :] = v
                return 0
            lax.fori_loop(0, EP, nr, 0)
            pltpu.sync_copy(ebuf, out_h.at[cid, pl.ds(r0, EP)])
            return 0
        lax.fori_loop(0, NROW // EP, ep, 0)

    return k


def _pad_rows(x):
    return jnp.pad(x, ((0, NPAD - N), (0, 0)))


def _gat_conv(xl, xr, src_p, dst_p, att, b, relu):
    C = xl.shape[1]
    C16 = C // 16
    HALF16 = C16 // 2
    G, GROUP, EP = (32, 24, 32) if C == 256 else (64, 24, 64)
    src3 = src_p.reshape(16, SLICE // G, G)
    dst3 = dst_p.reshape(16, SLICE // G, G)
    xl3 = _pad_rows(xl).reshape(NPAD, C16, 16)
    xr3 = _pad_rows(xr).reshape(NPAD, C16, 16)
    out = _gat_sc_kernel(C16, HALF16, G, GROUP, EP, True, relu)(
        src3, dst3, xl3, xr3, att.reshape(C16, 16), b.reshape(C16, 16))
    halves = out[:, :, :HALF16, :].reshape(2, NPAD, C // 2)
    return jnp.concatenate([halves[0], halves[1]], axis=-1)[:N]


def _gat_conv_pair(xlmu, xrmu, xlls, xrls, src_p, dst_p, attmu, bmu,
                   attls, bls):
    G = 128
    src3 = src_p.reshape(16, SLICE // G, G)
    dst3 = dst_p.reshape(16, SLICE // G, G)
    xl3 = _pad_rows(jnp.concatenate([xlmu, xlls], -1)).reshape(NPAD, 2, 16)
    xr3 = _pad_rows(jnp.concatenate([xrmu, xrls], -1)).reshape(NPAD, 2, 16)
    att2 = jnp.stack([attmu, attls])
    b2 = jnp.stack([bmu, bls])
    out = _gat_sc_kernel(2, 1, G, 84, 128, False, False)(
        src3, dst3, xl3, xr3, att2, b2)
    mu = out[0, :, 0, :].reshape(NPAD, 16)[:N]
    ls = out[1, :, 0, :].reshape(NPAD, 16)[:N]
    return mu, ls


def _matmul(x, w):
    M, K = x.shape
    Co = w.shape[1]
    BM = 1000

    def body(x_ref, w_ref, o_ref):
        o_ref[...] = jnp.dot(x_ref[...], w_ref[...],
                             preferred_element_type=jnp.float32)

    return pl.pallas_call(
        body,
        grid=(M // BM,),
        in_specs=[pl.BlockSpec((BM, K), lambda i: (i, 0)),
                  pl.BlockSpec((K, Co), lambda i: (0, 0))],
        out_specs=pl.BlockSpec((BM, Co), lambda i: (i, 0)),
        out_shape=jax.ShapeDtypeStruct((M, Co), jnp.float32),
    )(x, w)


def _leaky(x):
    return jnp.maximum(x, 0.0) + 0.2 * jnp.minimum(x, 0.0)


def _decoder(mu, logstd, eps, p):
    def body(mu_r, ls_r, eps_r, w1, b1, g1, be1, w2, b2, g2, be2, z_r, d_r):
        z = eps_r[...] * jnp.exp(ls_r[...]) + mu_r[...]
        z_r[...] = z
        a = jnp.dot(z, w1[...], preferred_element_type=jnp.float32) + b1[...]
        m = jnp.mean(a, axis=0, keepdims=True)
        v = jnp.mean((a - m) ** 2, axis=0, keepdims=True)
        a = _leaky(g1[...] * (a - m) / jnp.sqrt(v + 1e-5) + be1[...])
        d = jnp.dot(a, w2[...], preferred_element_type=jnp.float32) + b2[...]
        m2 = jnp.mean(d, axis=0, keepdims=True)
        v2 = jnp.mean((d - m2) ** 2, axis=0, keepdims=True)
        d_r[...] = _leaky(g2[...] * (d - m2) / jnp.sqrt(v2 + 1e-5) + be2[...])

    args = (mu, logstd, eps,
            p["W_d1"], p["b_d1"].reshape(1, -1),
            p["g1"].reshape(1, -1), p["be1"].reshape(1, -1),
            p["W_d2"], p["b_d2"].reshape(1, -1),
            p["g2"].reshape(1, -1), p["be2"].reshape(1, -1))
    return pl.pallas_call(
        body,
        out_shape=(jax.ShapeDtypeStruct((N, 16), jnp.float32),
                   jax.ShapeDtypeStruct((N, 256), jnp.float32)),
    )(*args)


def _adj(d):
    B = 400

    def body(a_ref, b_ref, o_ref):
        o_ref[...] = jax.nn.sigmoid(
            lax.dot_general(a_ref[...], b_ref[...],
                            (((1,), (1,)), ((), ())),
                            preferred_element_type=jnp.float32))

    return pl.pallas_call(
        body,
        grid=(N // B,),
        in_specs=[pl.BlockSpec((B, 256), lambda i: (i, 0)),
                  pl.BlockSpec((N, 256), lambda i: (0, 0))],
        out_specs=pl.BlockSpec((B, N), lambda i: (i, 0)),
        out_shape=jax.ShapeDtypeStruct((N, N), jnp.float32),
    )(d, d)


def _heads(d, p):
    BM = 1000
    OUT = p["W21"].shape[1]

    def body(d_ref, w1, c1, w2, c2, w3, c3, r1, r2, r3):
        x = d_ref[...]
        r1[...] = jnp.exp(
            jnp.dot(x, w1[...], preferred_element_type=jnp.float32) + c1[...])
        s = jnp.dot(x, w2[...], preferred_element_type=jnp.float32) + c2[...]
        r2[...] = jnp.clip(jax.nn.softplus(s), 1e-4, 1e4)
        r3[...] = jax.nn.sigmoid(
            jnp.dot(x, w3[...], preferred_element_type=jnp.float32) + c3[...])

    wspec = pl.BlockSpec((256, OUT), lambda i: (0, 0))
    bspec = pl.BlockSpec((1, OUT), lambda i: (0, 0))
    ospec = pl.BlockSpec((BM, OUT), lambda i: (i, 0))
    oshape = jax.ShapeDtypeStruct((N, OUT), jnp.float32)
    return pl.pallas_call(
        body,
        grid=(N // BM,),
        in_specs=[pl.BlockSpec((BM, 256), lambda i: (i, 0)),
                  wspec, bspec, wspec, bspec, wspec, bspec],
        out_specs=(ospec, ospec, ospec),
        out_shape=(oshape, oshape, oshape),
    )(d, p["W21"], p["b21"].reshape(1, -1),
      p["W22"], p["b22"].reshape(1, -1),
      p["W23"], p["b23"].reshape(1, -1))


def kernel(x, edge_index, spatial_coord, params):
    del spatial_coord
    loop = jnp.arange(N, dtype=edge_index.dtype)
    src_p = jnp.concatenate([
        edge_index[0], loop,
        jnp.zeros((ET_P - ET,), edge_index.dtype)])
    dst_p = jnp.concatenate([
        edge_index[1], loop,
        jnp.full((ET_P - ET,), NPAD - 1, edge_index.dtype)])

    def conv(h, name, relu):
        cp = params[name]
        C = cp["Wl"].shape[1]
        xlxr = _matmul(h, jnp.concatenate([cp["Wl"], cp["Wr"]], axis=1))
        return _gat_conv(xlxr[:, :C], xlxr[:, C:], src_p, dst_p,
                         cp["att"], cp["b"], relu)

    h = conv(x, "conv1", True)
    h = conv(h, "conv2", True)

    pmu, pls = params["conv_mu"], params["conv_logstd"]
    w4 = jnp.concatenate([pmu["Wl"], pmu["Wr"], pls["Wl"], pls["Wr"]], 1)
    xx = _matmul(h, w4)
    mu, logstd = _gat_conv_pair(
        xx[:, 0:16], xx[:, 16:32], xx[:, 32:48], xx[:, 48:64],
        src_p, dst_p, pmu["att"], pmu["b"], pls["att"], pls["b"])

    eps = jax.random.normal(jax.random.key(7), (N, 16), dtype=jnp.float32)
    z, d = _decoder(mu, logstd, eps, params)
    adj_pred = _adj(d)
    r1, r2, r3 = _heads(d, params)
    return (z, r1, r2, r3, mu, logstd, adj_pred)


# speedup vs baseline: 4.1924x; 1.2285x over previous
import functools

import jax
import jax.numpy as jnp
from jax import lax
from jax.experimental import pallas as pl
from jax.experimental.pallas import tpu as pltpu
from jax.experimental.pallas import tpu_sc as plsc

N = 10000
E = 160000
ET = E + N
NPAD = 10240
NW = 32
ECHUNK = 2000
NCH = ET // ECHUNK
LCAP = ECHUNK + 48

_GDN = lax.GatherDimensionNumbers(
    offset_dims=(), collapsed_slice_dims=(0,), start_index_map=(0,))


def _permute(v, idx):
    return lax.gather(v, idx[:, None], _GDN, slice_sizes=(1,),
                      mode=lax.GatherScatterMode.PROMISE_IN_BOUNDS)


def _lane_iota():
    return lax.iota(jnp.int32, 16)


def _allsum(v):
    lane = _lane_iota()
    s = v
    for k in (1, 2, 4, 8):
        s = s + _permute(s, jnp.bitwise_xor(lane, k))
    return s


def _inclusive_prefix(v):
    lane = _lane_iota()
    s = v
    for k in (1, 2, 4, 8):
        shifted = _permute(s, jnp.maximum(lane - k, 0))
        s = s + jnp.where(lane >= k, shifted, 0)
    return s


SLICE = 10752
ET_P = 16 * SLICE


def _gat_sc_kernel(C16, HALF16, G, GROUP, EP, efull, relu):
    NBAT = SLICE // G
    NGROUP = NBAT // GROUP
    NROW = NPAD // 16
    assert NBAT % GROUP == 0 and NROW % EP == 0

    mesh = plsc.VectorSubcoreMesh(core_axis_name="c", subcore_axis_name="s")

    @functools.partial(
        pl.kernel,
        mesh=mesh,
        compiler_params=pltpu.CompilerParams(use_tc_tiling_on_sc=False),
        out_type=jax.ShapeDtypeStruct((2, NPAD, HALF16 + 1, 16),
                                      jnp.float32),
        scratch_types=[
            pltpu.VMEM((GROUP, G), jnp.int32),
            pltpu.VMEM((GROUP, G), jnp.int32),
            pltpu.VMEM((G, C16, 16), jnp.float32),
            pltpu.VMEM((G, C16, 16), jnp.float32),
            pltpu.VMEM((G, C16, 16), jnp.float32),
            pltpu.VMEM((G, C16, 16), jnp.float32),
            pltpu.VMEM((G, HALF16 + 1, 16), jnp.float32),
            pltpu.VMEM((EP, HALF16 + 1, 16), jnp.float32),
            pltpu.VMEM((C16, 16), jnp.float32),
            pltpu.VMEM((C16, 16), jnp.float32),
            pltpu.VMEM_SHARED((NPAD, HALF16 + 1, 16), jnp.float32),
            pltpu.SemaphoreType.DMA,
            pltpu.SemaphoreType.DMA,
        ],
    )
    def k(src_h, dst_h, xl_h, xr_h, att_h, b_h, out_h,
          sidx, didx, xlg0, xrg0, xlg1, xrg1, ctb, ebuf, attv, bv, acc_sh,
          sem0, sem1):
        cid = lax.axis_index("c")
        sid = lax.axis_index("s")
        pltpu.sync_copy(att_h, attv)
        pltpu.sync_copy(b_h, bv)
        zf = jnp.zeros((16,), jnp.float32)

        def _zc(i, _):
            for c in range(HALF16 + 1):
                ctb[i, c, :] = zf
            return 0
        lax.fori_loop(0, G, _zc, 0)

        def _za(i, _):
            pltpu.sync_copy(ctb, acc_sh.at[pl.ds(sid * NROW + i * G, G)])
            return 0
        lax.fori_loop(0, NROW // G, _za, 0)
        plsc.subcore_barrier()

        slots = ((xlg0, xrg0, sem0), (xlg1, xrg1, sem1))

        def _start(bi, slot):
            xg, rg, sm = slots[slot]
            pltpu.async_copy(xl_h.at[sidx.at[bi]], xg, sm)
            pltpu.async_copy(xr_h.at[didx.at[bi]], rg, sm)

        def _process(bi, slot):
            xlg, xrg, sm = slots[slot]
            pltpu.make_async_copy(xl_h.at[sidx.at[bi]], xlg, sm).wait()
            pltpu.make_async_copy(xr_h.at[didx.at[bi]], xrg, sm).wait()

            @pl.when(bi + 1 < GROUP)
            def _():
                _start(bi + 1, 1 - slot)

            def edge(j, _):
                ea = zf
                if efull:
                    for c in range(C16):
                        sv = xlg[j, c, :] + xrg[j, c, :]
                        lk = (jnp.maximum(sv, 0.0)
                              + 0.2 * jnp.minimum(sv, 0.0))
                        ea = ea + attv[c, :] * lk
                else:
                    for c in range(HALF16):
                        c2 = cid * HALF16 + c
                        sv = xlg[j, c2, :] + xrg[j, c2, :]
                        lk = (jnp.maximum(sv, 0.0)
                              + 0.2 * jnp.minimum(sv, 0.0))
                        ea = ea + attv[c2, :] * lk
                e = _allsum(ea)
                e = jnp.minimum(jnp.maximum(e, -60.0), 60.0)
                ex = jnp.exp(e)
                for c in range(HALF16):
                    c2 = cid * HALF16 + c
                    ctb[j, c, :] = ex * xlg[j, c2, :]
                ctb[j, HALF16, :] = ex
                return 0
            lax.fori_loop(0, G, edge, 0)
            pltpu.sync_copy(ctb, acc_sh.at[didx.at[bi]], add=True)

        def group_body(g, _):
            pltpu.sync_copy(src_h.at[sid, pl.ds(g * GROUP, GROUP)], sidx)
            pltpu.sync_copy(dst_h.at[sid, pl.ds(g * GROUP, GROUP)], didx)
            _start(0, 0)

            def pair(p, _):
                _process(p * 2, 0)
                _process(p * 2 + 1, 1)
                return 0
            lax.fori_loop(0, GROUP // 2, pair, 0)
            return 0
        lax.fori_loop(0, NGROUP, group_body, 0)
        plsc.subcore_barrier()

        def ep(i, _):
            r0 = sid * NROW + i * EP
            pltpu.sync_copy(acc_sh.at[pl.ds(r0, EP)], ebuf)

            def nr(r, _):
                inv = 1.0 / (ebuf[r, HALF16, :] + 1e-16)
                for c in range(HALF16):
                    c2 = cid * HALF16 + c
                    v = ebuf[r, c, :] * inv + bv[c2, :]
                    if relu:
                        v = jnp.maximum(v, 0.0)
                    ebuf[r, c, :] = v
                return 0
            lax.fori_loop(0, EP, nr, 0)
            pltpu.sync_copy(ebuf, out_h.at[cid, pl.ds(r0, EP)])
            return 0
        lax.fori_loop(0, NROW // EP, ep, 0)

    return k


def _pad_rows(x):
    return jnp.pad(x, ((0, NPAD - N), (0, 0)))


def _gat_conv(xl, xr, src_p, dst_p, att, b, relu):
    C = xl.shape[1]
    C16 = C // 16
    HALF16 = C16 // 2
    G, GROUP, EP = (24, 28, 32) if C == 256 else (64, 24, 64)
    src3 = src_p.reshape(16, SLICE // G, G)
    dst3 = dst_p.reshape(16, SLICE // G, G)
    xl3 = _pad_rows(xl).reshape(NPAD, C16, 16)
    xr3 = _pad_rows(xr).reshape(NPAD, C16, 16)
    out = _gat_sc_kernel(C16, HALF16, G, GROUP, EP, True, relu)(
        src3, dst3, xl3, xr3, att.reshape(C16, 16), b.reshape(C16, 16))
    halves = out[:, :, :HALF16, :].reshape(2, NPAD, C // 2)
    return jnp.concatenate([halves[0], halves[1]], axis=-1)[:N]


def _gat_conv_pair(xlmu, xrmu, xlls, xrls, src_p, dst_p, attmu, bmu,
                   attls, bls):
    G = 128
    src3 = src_p.reshape(16, SLICE // G, G)
    dst3 = dst_p.reshape(16, SLICE // G, G)
    xl3 = _pad_rows(jnp.concatenate([xlmu, xlls], -1)).reshape(NPAD, 2, 16)
    xr3 = _pad_rows(jnp.concatenate([xrmu, xrls], -1)).reshape(NPAD, 2, 16)
    att2 = jnp.stack([attmu, attls])
    b2 = jnp.stack([bmu, bls])
    out = _gat_sc_kernel(2, 1, G, 84, 128, False, False)(
        src3, dst3, xl3, xr3, att2, b2)
    mu = out[0, :, 0, :].reshape(NPAD, 16)[:N]
    ls = out[1, :, 0, :].reshape(NPAD, 16)[:N]
    return mu, ls


def _matmul(x, w):
    M, K = x.shape
    Co = w.shape[1]
    BM = 1000

    def body(x_ref, w_ref, o_ref):
        o_ref[...] = jnp.dot(x_ref[...], w_ref[...],
                             preferred_element_type=jnp.float32)

    return pl.pallas_call(
        body,
        grid=(M // BM,),
        in_specs=[pl.BlockSpec((BM, K), lambda i: (i, 0)),
                  pl.BlockSpec((K, Co), lambda i: (0, 0))],
        out_specs=pl.BlockSpec((BM, Co), lambda i: (i, 0)),
        out_shape=jax.ShapeDtypeStruct((M, Co), jnp.float32),
    )(x, w)


def _leaky(x):
    return jnp.maximum(x, 0.0) + 0.2 * jnp.minimum(x, 0.0)


def _decoder(mu, logstd, eps, p):
    def body(mu_r, ls_r, eps_r, w1, b1, g1, be1, w2, b2, g2, be2, z_r, d_r):
        z = eps_r[...] * jnp.exp(ls_r[...]) + mu_r[...]
        z_r[...] = z
        a = jnp.dot(z, w1[...], preferred_element_type=jnp.float32) + b1[...]
        m = jnp.mean(a, axis=0, keepdims=True)
        v = jnp.mean((a - m) ** 2, axis=0, keepdims=True)
        a = _leaky(g1[...] * (a - m) / jnp.sqrt(v + 1e-5) + be1[...])
        d = jnp.dot(a, w2[...], preferred_element_type=jnp.float32) + b2[...]
        m2 = jnp.mean(d, axis=0, keepdims=True)
        v2 = jnp.mean((d - m2) ** 2, axis=0, keepdims=True)
        d_r[...] = _leaky(g2[...] * (d - m2) / jnp.sqrt(v2 + 1e-5) + be2[...])

    args = (mu, logstd, eps,
            p["W_d1"], p["b_d1"].reshape(1, -1),
            p["g1"].reshape(1, -1), p["be1"].reshape(1, -1),
            p["W_d2"], p["b_d2"].reshape(1, -1),
            p["g2"].reshape(1, -1), p["be2"].reshape(1, -1))
    return pl.pallas_call(
        body,
        out_shape=(jax.ShapeDtypeStruct((N, 16), jnp.float32),
                   jax.ShapeDtypeStruct((N, 256), jnp.float32)),
    )(*args)


def _adj(d):
    B = 400

    def body(a_ref, b_ref, o_ref):
        o_ref[...] = jax.nn.sigmoid(
            lax.dot_general(a_ref[...], b_ref[...],
                            (((1,), (1,)), ((), ())),
                            preferred_element_type=jnp.float32))

    return pl.pallas_call(
        body,
        grid=(N // B,),
        in_specs=[pl.BlockSpec((B, 256), lambda i: (i, 0)),
                  pl.BlockSpec((N, 256), lambda i: (0, 0))],
        out_specs=pl.BlockSpec((B, N), lambda i: (i, 0)),
        out_shape=jax.ShapeDtypeStruct((N, N), jnp.float32),
    )(d, d)


def _heads(d, p):
    BM = 1000
    OUT = p["W21"].shape[1]

    def body(d_ref, w1, c1, w2, c2, w3, c3, r1, r2, r3):
        x = d_ref[...]
        r1[...] = jnp.exp(
            jnp.dot(x, w1[...], preferred_element_type=jnp.float32) + c1[...])
        s = jnp.dot(x, w2[...], preferred_element_type=jnp.float32) + c2[...]
        r2[...] = jnp.clip(jax.nn.softplus(s), 1e-4, 1e4)
        r3[...] = jax.nn.sigmoid(
            jnp.dot(x, w3[...], preferred_element_type=jnp.float32) + c3[...])

    wspec = pl.BlockSpec((256, OUT), lambda i: (0, 0))
    bspec = pl.BlockSpec((1, OUT), lambda i: (0, 0))
    ospec = pl.BlockSpec((BM, OUT), lambda i: (i, 0))
    oshape = jax.ShapeDtypeStruct((N, OUT), jnp.float32)
    return pl.pallas_call(
        body,
        grid=(N // BM,),
        in_specs=[pl.BlockSpec((BM, 256), lambda i: (i, 0)),
                  wspec, bspec, wspec, bspec, wspec, bspec],
        out_specs=(ospec, ospec, ospec),
        out_shape=(oshape, oshape, oshape),
    )(d, p["W21"], p["b21"].reshape(1, -1),
      p["W22"], p["b22"].reshape(1, -1),
      p["W23"], p["b23"].reshape(1, -1))


def kernel(x, edge_index, spatial_coord, params):
    del spatial_coord
    loop = jnp.arange(N, dtype=edge_index.dtype)
    src_p = jnp.concatenate([
        edge_index[0], loop,
        jnp.zeros((ET_P - ET,), edge_index.dtype)])
    dst_p = jnp.concatenate([
        edge_index[1], loop,
        jnp.full((ET_P - ET,), NPAD - 1, edge_index.dtype)])

    def conv(h, name, relu):
        cp = params[name]
        C = cp["Wl"].shape[1]
        xlxr = _matmul(h, jnp.concatenate([cp["Wl"], cp["Wr"]], axis=1))
        return _gat_conv(xlxr[:, :C], xlxr[:, C:], src_p, dst_p,
                         cp["att"], cp["b"], relu)

    h = conv(x, "conv1", True)
    h = conv(h, "conv2", True)

    pmu, pls = params["conv_mu"], params["conv_logstd"]
    w4 = jnp.concatenate([pmu["Wl"], pmu["Wr"], pls["Wl"], pls["Wr"]], 1)
    xx = _matmul(h, w4)
    mu, logstd = _gat_conv_pair(
        xx[:, 0:16], xx[:, 16:32], xx[:, 32:48], xx[:, 48:64],
        src_p, dst_p, pmu["att"], pmu["b"], pls["att"], pls["b"])

    eps = jax.random.normal(jax.random.key(7), (N, 16), dtype=jnp.float32)
    z, d = _decoder(mu, logstd, eps, params)
    adj_pred = _adj(d)
    r1, r2, r3 = _heads(d, params)
    return (z, r1, r2, r3, mu, logstd, adj_pred)


# speedup vs baseline: 4.6294x; 1.1042x over previous
import functools

import jax
import jax.numpy as jnp
from jax import lax
from jax.experimental import pallas as pl
from jax.experimental.pallas import tpu as pltpu
from jax.experimental.pallas import tpu_sc as plsc

N = 10000
E = 160000
ET = E + N
NPAD = 10240
NW = 32
ECHUNK = 2000
NCH = ET // ECHUNK
LCAP = ECHUNK + 48

_GDN = lax.GatherDimensionNumbers(
    offset_dims=(), collapsed_slice_dims=(0,), start_index_map=(0,))


def _permute(v, idx):
    return lax.gather(v, idx[:, None], _GDN, slice_sizes=(1,),
                      mode=lax.GatherScatterMode.PROMISE_IN_BOUNDS)


def _lane_iota():
    return lax.iota(jnp.int32, 16)


def _allsum(v):
    lane = _lane_iota()
    s = v
    for k in (1, 2, 4, 8):
        s = s + _permute(s, jnp.bitwise_xor(lane, k))
    return s


def _inclusive_prefix(v):
    lane = _lane_iota()
    s = v
    for k in (1, 2, 4, 8):
        shifted = _permute(s, jnp.maximum(lane - k, 0))
        s = s + jnp.where(lane >= k, shifted, 0)
    return s


SLICE = 10752
ET_P = 16 * SLICE


def _gat_sc_kernel(C16, HALF16, G, GROUP, EP, efull, relu):
    NBAT = SLICE // G
    NGROUP = NBAT // GROUP
    NROW = NPAD // 16
    assert NBAT % GROUP == 0 and NROW % EP == 0

    mesh = plsc.VectorSubcoreMesh(core_axis_name="c", subcore_axis_name="s")

    @functools.partial(
        pl.kernel,
        mesh=mesh,
        compiler_params=pltpu.CompilerParams(use_tc_tiling_on_sc=False),
        out_type=jax.ShapeDtypeStruct((2, NPAD, HALF16 + 1, 16),
                                      jnp.float32),
        scratch_types=[
            pltpu.VMEM((GROUP, G), jnp.int32),
            pltpu.VMEM((GROUP, G), jnp.int32),
            pltpu.VMEM((G, C16, 16), jnp.float32),
            pltpu.VMEM((G, C16, 16), jnp.float32),
            pltpu.VMEM((G, C16, 16), jnp.float32),
            pltpu.VMEM((G, C16, 16), jnp.float32),
            pltpu.VMEM((G, HALF16 + 1, 16), jnp.float32),
            pltpu.VMEM((EP, HALF16 + 1, 16), jnp.float32),
            pltpu.VMEM((C16, 16), jnp.float32),
            pltpu.VMEM((C16, 16), jnp.float32),
            pltpu.VMEM_SHARED((NPAD, HALF16 + 1, 16), jnp.float32),
            pltpu.SemaphoreType.DMA,
            pltpu.SemaphoreType.DMA,
        ],
    )
    def k(src_h, dst_h, xl_h, xr_h, att_h, b_h, out_h,
          sidx, didx, xlg0, xrg0, xlg1, xrg1, ctb, ebuf, attv, bv, acc_sh,
          sem0, sem1):
        cid = lax.axis_index("c")
        sid = lax.axis_index("s")
        pltpu.sync_copy(att_h, attv)
        pltpu.sync_copy(b_h, bv)
        zf = jnp.zeros((16,), jnp.float32)
        cid_f = jnp.broadcast_to(cid, (16,)).astype(jnp.float32)

        def _zc(i, _):
            for c in range(HALF16 + 1):
                ctb[i, c, :] = zf
            return 0
        lax.fori_loop(0, G, _zc, 0)

        def _za(i, _):
            pltpu.sync_copy(ctb, acc_sh.at[pl.ds(sid * NROW + i * G, G)])
            return 0
        lax.fori_loop(0, NROW // G, _za, 0)
        plsc.subcore_barrier()

        slots = ((xlg0, xrg0, sem0), (xlg1, xrg1, sem1))

        def _start(bi, slot):
            xg, rg, sm = slots[slot]
            pltpu.async_copy(xl_h.at[sidx.at[bi]], xg, sm)
            pltpu.async_copy(xr_h.at[didx.at[bi]], rg, sm)

        def _process(bi, slot):
            xlg, xrg, sm = slots[slot]
            pltpu.make_async_copy(xl_h.at[sidx.at[bi]], xlg, sm).wait()
            pltpu.make_async_copy(xr_h.at[didx.at[bi]], xrg, sm).wait()

            @pl.when(bi + 1 < GROUP)
            def _():
                _start(bi + 1, 1 - slot)

            def edge(j, _):
                ea0 = zf
                ea1 = zf
                xs = {}
                if efull:
                    for c in range(C16):
                        xv = xlg[j, c, :]
                        xs[c] = xv
                        sv = xv + xrg[j, c, :]
                        lk = (jnp.maximum(sv, 0.0)
                              + 0.2 * jnp.minimum(sv, 0.0))
                        if c % 2 == 0:
                            ea0 = ea0 + attv[c, :] * lk
                        else:
                            ea1 = ea1 + attv[c, :] * lk
                else:
                    for c in range(HALF16):
                        c2 = cid * HALF16 + c
                        xv = xlg[j, c2, :]
                        xs[c] = xv
                        sv = xv + xrg[j, c2, :]
                        lk = (jnp.maximum(sv, 0.0)
                              + 0.2 * jnp.minimum(sv, 0.0))
                        ea0 = ea0 + attv[c2, :] * lk
                e = _allsum(ea0 + ea1)
                e = jnp.minimum(jnp.maximum(e, -60.0), 60.0)
                ex = jnp.exp(e)
                for c in range(HALF16):
                    if efull:
                        own = xs[c] + (xs[HALF16 + c] - xs[c]) * cid_f
                        ctb[j, c, :] = ex * own
                    else:
                        ctb[j, c, :] = ex * xs[c]
                ctb[j, HALF16, :] = ex
                return 0
            lax.fori_loop(0, G, edge, 0)
            pltpu.sync_copy(ctb, acc_sh.at[didx.at[bi]], add=True)

        def group_body(g, _):
            pltpu.sync_copy(src_h.at[sid, pl.ds(g * GROUP, GROUP)], sidx)
            pltpu.sync_copy(dst_h.at[sid, pl.ds(g * GROUP, GROUP)], didx)
            _start(0, 0)

            def pair(p, _):
                _process(p * 2, 0)
                _process(p * 2 + 1, 1)
                return 0
            lax.fori_loop(0, GROUP // 2, pair, 0)
            return 0
        lax.fori_loop(0, NGROUP, group_body, 0)
        plsc.subcore_barrier()

        def ep(i, _):
            r0 = sid * NROW + i * EP
            pltpu.sync_copy(acc_sh.at[pl.ds(r0, EP)], ebuf)

            def nr(r, _):
                inv = 1.0 / (ebuf[r, HALF16, :] + 1e-16)
                for c in range(HALF16):
                    c2 = cid * HALF16 + c
                    v = ebuf[r, c, :] * inv + bv[c2, :]
                    if relu:
                        v = jnp.maximum(v, 0.0)
                    ebuf[r, c, :] = v
                return 0
            lax.fori_loop(0, EP, nr, 0)
            pltpu.sync_copy(ebuf, out_h.at[cid, pl.ds(r0, EP)])
            return 0
        lax.fori_loop(0, NROW // EP, ep, 0)

    return k


def _pad_rows(x):
    return jnp.pad(x, ((0, NPAD - N), (0, 0)))


def _gat_conv(xl, xr, src_p, dst_p, att, b, relu):
    C = xl.shape[1]
    C16 = C // 16
    HALF16 = C16 // 2
    G, GROUP, EP = (24, 28, 32) if C == 256 else (64, 24, 64)
    src3 = src_p.reshape(16, SLICE // G, G)
    dst3 = dst_p.reshape(16, SLICE // G, G)
    xl3 = _pad_rows(xl).reshape(NPAD, C16, 16)
    xr3 = _pad_rows(xr).reshape(NPAD, C16, 16)
    out = _gat_sc_kernel(C16, HALF16, G, GROUP, EP, True, relu)(
        src3, dst3, xl3, xr3, att.reshape(C16, 16), b.reshape(C16, 16))
    halves = out[:, :, :HALF16, :].reshape(2, NPAD, C // 2)
    return jnp.concatenate([halves[0], halves[1]], axis=-1)[:N]


def _gat_conv_pair(xlmu, xrmu, xlls, xrls, src_p, dst_p, attmu, bmu,
                   attls, bls):
    G = 128
    src3 = src_p.reshape(16, SLICE // G, G)
    dst3 = dst_p.reshape(16, SLICE // G, G)
    xl3 = _pad_rows(jnp.concatenate([xlmu, xlls], -1)).reshape(NPAD, 2, 16)
    xr3 = _pad_rows(jnp.concatenate([xrmu, xrls], -1)).reshape(NPAD, 2, 16)
    att2 = jnp.stack([attmu, attls])
    b2 = jnp.stack([bmu, bls])
    out = _gat_sc_kernel(2, 1, G, 84, 128, False, False)(
        src3, dst3, xl3, xr3, att2, b2)
    mu = out[0, :, 0, :].reshape(NPAD, 16)[:N]
    ls = out[1, :, 0, :].reshape(NPAD, 16)[:N]
    return mu, ls


def _matmul(x, w):
    M, K = x.shape
    Co = w.shape[1]
    BM = 1000

    def body(x_ref, w_ref, o_ref):
        o_ref[...] = jnp.dot(x_ref[...], w_ref[...],
                             preferred_element_type=jnp.float32)

    return pl.pallas_call(
        body,
        grid=(M // BM,),
        in_specs=[pl.BlockSpec((BM, K), lambda i: (i, 0)),
                  pl.BlockSpec((K, Co), lambda i: (0, 0))],
        out_specs=pl.BlockSpec((BM, Co), lambda i: (i, 0)),
        out_shape=jax.ShapeDtypeStruct((M, Co), jnp.float32),
    )(x, w)


def _leaky(x):
    return jnp.maximum(x, 0.0) + 0.2 * jnp.minimum(x, 0.0)


def _decoder(mu, logstd, eps, p):
    def body(mu_r, ls_r, eps_r, w1, b1, g1, be1, w2, b2, g2, be2, z_r, d_r):
        z = eps_r[...] * jnp.exp(ls_r[...]) + mu_r[...]
        z_r[...] = z
        a = jnp.dot(z, w1[...], preferred_element_type=jnp.float32) + b1[...]
        m = jnp.mean(a, axis=0, keepdims=True)
        v = jnp.mean((a - m) ** 2, axis=0, keepdims=True)
        a = _leaky(g1[...] * (a - m) / jnp.sqrt(v + 1e-5) + be1[...])
        d = jnp.dot(a, w2[...], preferred_element_type=jnp.float32) + b2[...]
        m2 = jnp.mean(d, axis=0, keepdims=True)
        v2 = jnp.mean((d - m2) ** 2, axis=0, keepdims=True)
        d_r[...] = _leaky(g2[...] * (d - m2) / jnp.sqrt(v2 + 1e-5) + be2[...])

    args = (mu, logstd, eps,
            p["W_d1"], p["b_d1"].reshape(1, -1),
            p["g1"].reshape(1, -1), p["be1"].reshape(1, -1),
            p["W_d2"], p["b_d2"].reshape(1, -1),
            p["g2"].reshape(1, -1), p["be2"].reshape(1, -1))
    return pl.pallas_call(
        body,
        out_shape=(jax.ShapeDtypeStruct((N, 16), jnp.float32),
                   jax.ShapeDtypeStruct((N, 256), jnp.float32)),
    )(*args)


def _adj(d):
    B = 400

    def body(a_ref, b_ref, o_ref):
        o_ref[...] = jax.nn.sigmoid(
            lax.dot_general(a_ref[...], b_ref[...],
                            (((1,), (1,)), ((), ())),
                            preferred_element_type=jnp.float32))

    return pl.pallas_call(
        body,
        grid=(N // B,),
        in_specs=[pl.BlockSpec((B, 256), lambda i: (i, 0)),
                  pl.BlockSpec((N, 256), lambda i: (0, 0))],
        out_specs=pl.BlockSpec((B, N), lambda i: (i, 0)),
        out_shape=jax.ShapeDtypeStruct((N, N), jnp.float32),
    )(d, d)


def _heads(d, p):
    BM = 1000
    OUT = p["W21"].shape[1]

    def body(d_ref, w1, c1, w2, c2, w3, c3, r1, r2, r3):
        x = d_ref[...]
        r1[...] = jnp.exp(
            jnp.dot(x, w1[...], preferred_element_type=jnp.float32) + c1[...])
        s = jnp.dot(x, w2[...], preferred_element_type=jnp.float32) + c2[...]
        r2[...] = jnp.clip(jax.nn.softplus(s), 1e-4, 1e4)
        r3[...] = jax.nn.sigmoid(
            jnp.dot(x, w3[...], preferred_element_type=jnp.float32) + c3[...])

    wspec = pl.BlockSpec((256, OUT), lambda i: (0, 0))
    bspec = pl.BlockSpec((1, OUT), lambda i: (0, 0))
    ospec = pl.BlockSpec((BM, OUT), lambda i: (i, 0))
    oshape = jax.ShapeDtypeStruct((N, OUT), jnp.float32)
    return pl.pallas_call(
        body,
        grid=(N // BM,),
        in_specs=[pl.BlockSpec((BM, 256), lambda i: (i, 0)),
                  wspec, bspec, wspec, bspec, wspec, bspec],
        out_specs=(ospec, ospec, ospec),
        out_shape=(oshape, oshape, oshape),
    )(d, p["W21"], p["b21"].reshape(1, -1),
      p["W22"], p["b22"].reshape(1, -1),
      p["W23"], p["b23"].reshape(1, -1))


def kernel(x, edge_index, spatial_coord, params):
    del spatial_coord
    loop = jnp.arange(N, dtype=edge_index.dtype)
    src_p = jnp.concatenate([
        edge_index[0], loop,
        jnp.zeros((ET_P - ET,), edge_index.dtype)])
    dst_p = jnp.concatenate([
        edge_index[1], loop,
        jnp.full((ET_P - ET,), NPAD - 1, edge_index.dtype)])

    def conv(h, name, relu):
        cp = params[name]
        C = cp["Wl"].shape[1]
        xlxr = _matmul(h, jnp.concatenate([cp["Wl"], cp["Wr"]], axis=1))
        return _gat_conv(xlxr[:, :C], xlxr[:, C:], src_p, dst_p,
                         cp["att"], cp["b"], relu)

    h = conv(x, "conv1", True)
    h = conv(h, "conv2", True)

    pmu, pls = params["conv_mu"], params["conv_logstd"]
    w4 = jnp.concatenate([pmu["Wl"], pmu["Wr"], pls["Wl"], pls["Wr"]], 1)
    xx = _matmul(h, w4)
    mu, logstd = _gat_conv_pair(
        xx[:, 0:16], xx[:, 16:32], xx[:, 32:48], xx[:, 48:64],
        src_p, dst_p, pmu["att"], pmu["b"], pls["att"], pls["b"])

    eps = jax.random.normal(jax.random.key(7), (N, 16), dtype=jnp.float32)
    z, d = _decoder(mu, logstd, eps, params)
    adj_pred = _adj(d)
    r1, r2, r3 = _heads(d, params)
    return (z, r1, r2, r3, mu, logstd, adj_pred)


# speedup vs baseline: 4.6349x; 1.0012x over previous
import functools

import jax
import jax.numpy as jnp
from jax import lax
from jax.experimental import pallas as pl
from jax.experimental.pallas import tpu as pltpu
from jax.experimental.pallas import tpu_sc as plsc

N = 10000
E = 160000
ET = E + N
NPAD = 10240
NW = 32
ECHUNK = 2000
NCH = ET // ECHUNK
LCAP = ECHUNK + 48

_GDN = lax.GatherDimensionNumbers(
    offset_dims=(), collapsed_slice_dims=(0,), start_index_map=(0,))


def _permute(v, idx):
    return lax.gather(v, idx[:, None], _GDN, slice_sizes=(1,),
                      mode=lax.GatherScatterMode.PROMISE_IN_BOUNDS)


def _lane_iota():
    return lax.iota(jnp.int32, 16)


def _allsum(v):
    lane = _lane_iota()
    s = v
    for k in (1, 2, 4, 8):
        s = s + _permute(s, jnp.bitwise_xor(lane, k))
    return s


def _inclusive_prefix(v):
    lane = _lane_iota()
    s = v
    for k in (1, 2, 4, 8):
        shifted = _permute(s, jnp.maximum(lane - k, 0))
        s = s + jnp.where(lane >= k, shifted, 0)
    return s


SLICE = 10752
ET_P = 16 * SLICE


def _gat_sc_kernel(C16, HALF16, G, GROUP, EP, efull, relu):
    NBAT = SLICE // G
    NGROUP = NBAT // GROUP
    NROW = NPAD // 16
    assert NBAT % GROUP == 0 and NROW % EP == 0

    mesh = plsc.VectorSubcoreMesh(core_axis_name="c", subcore_axis_name="s")

    @functools.partial(
        pl.kernel,
        mesh=mesh,
        compiler_params=pltpu.CompilerParams(use_tc_tiling_on_sc=False),
        out_type=jax.ShapeDtypeStruct((2, NPAD, HALF16 + 1, 16),
                                      jnp.float32),
        scratch_types=[
            pltpu.VMEM((GROUP, G), jnp.int32),
            pltpu.VMEM((GROUP, G), jnp.int32),
            pltpu.VMEM((G, C16, 16), jnp.float32),
            pltpu.VMEM((G, C16, 16), jnp.float32),
            pltpu.VMEM((G, C16, 16), jnp.float32),
            pltpu.VMEM((G, C16, 16), jnp.float32),
            pltpu.VMEM((G, HALF16 + 1, 16), jnp.float32),
            pltpu.VMEM((EP, HALF16 + 1, 16), jnp.float32),
            pltpu.VMEM((C16, 16), jnp.float32),
            pltpu.VMEM((C16, 16), jnp.float32),
            pltpu.VMEM_SHARED((NPAD, HALF16 + 1, 16), jnp.float32),
            pltpu.SemaphoreType.DMA,
            pltpu.SemaphoreType.DMA,
        ],
    )
    def k(src_h, dst_h, xl_h, xr_h, att_h, b_h, out_h,
          sidx, didx, xlg0, xrg0, xlg1, xrg1, ctb, ebuf, attv, bv, acc_sh,
          sem0, sem1):
        cid = lax.axis_index("c")
        sid = lax.axis_index("s")
        pltpu.sync_copy(att_h, attv)
        pltpu.sync_copy(b_h, bv)
        zf = jnp.zeros((16,), jnp.float32)
        cid_f = jnp.broadcast_to(cid, (16,)).astype(jnp.float32)

        def _zc(i, _):
            for c in range(HALF16 + 1):
                ctb[i, c, :] = zf
            return 0
        lax.fori_loop(0, G, _zc, 0)

        def _za(i, _):
            pltpu.sync_copy(ctb, acc_sh.at[pl.ds(sid * NROW + i * G, G)])
            return 0
        lax.fori_loop(0, NROW // G, _za, 0)
        plsc.subcore_barrier()

        slots = ((xlg0, xrg0, sem0), (xlg1, xrg1, sem1))

        def _start(bi, slot):
            xg, rg, sm = slots[slot]
            pltpu.async_copy(xl_h.at[sidx.at[bi]], xg, sm)
            pltpu.async_copy(xr_h.at[didx.at[bi]], rg, sm)

        def _process(bi, slot):
            xlg, xrg, sm = slots[slot]
            pltpu.make_async_copy(xl_h.at[sidx.at[bi]], xlg, sm).wait()
            pltpu.make_async_copy(xr_h.at[didx.at[bi]], xrg, sm).wait()

            @pl.when(bi + 1 < GROUP)
            def _():
                _start(bi + 1, 1 - slot)

            def edge(j, _):
                ea0 = zf
                ea1 = zf
                xs = {}
                if efull:
                    for c in range(C16):
                        xv = xlg[j, c, :]
                        xs[c] = xv
                        sv = xv + xrg[j, c, :]
                        lk = (jnp.maximum(sv, 0.0)
                              + 0.2 * jnp.minimum(sv, 0.0))
                        if c % 2 == 0:
                            ea0 = ea0 + attv[c, :] * lk
                        else:
                            ea1 = ea1 + attv[c, :] * lk
                else:
                    for c in range(HALF16):
                        c2 = cid * HALF16 + c
                        xv = xlg[j, c2, :]
                        xs[c] = xv
                        sv = xv + xrg[j, c2, :]
                        lk = (jnp.maximum(sv, 0.0)
                              + 0.2 * jnp.minimum(sv, 0.0))
                        ea0 = ea0 + attv[c2, :] * lk
                e = _allsum(ea0 + ea1)
                e = jnp.minimum(jnp.maximum(e, -60.0), 60.0)
                ex = jnp.exp(e)
                for c in range(HALF16):
                    if efull:
                        own = xs[c] + (xs[HALF16 + c] - xs[c]) * cid_f
                        ctb[j, c, :] = ex * own
                    else:
                        ctb[j, c, :] = ex * xs[c]
                ctb[j, HALF16, :] = ex
                return 0
            lax.fori_loop(0, G, edge, 0)
            pltpu.sync_copy(ctb, acc_sh.at[didx.at[bi]], add=True)

        def group_body(g, _):
            pltpu.sync_copy(src_h.at[sid, pl.ds(g * GROUP, GROUP)], sidx)
            pltpu.sync_copy(dst_h.at[sid, pl.ds(g * GROUP, GROUP)], didx)
            _start(0, 0)

            def pair(p, _):
                _process(p * 2, 0)
                _process(p * 2 + 1, 1)
                return 0
            lax.fori_loop(0, GROUP // 2, pair, 0)
            return 0
        lax.fori_loop(0, NGROUP, group_body, 0)
        plsc.subcore_barrier()

        def ep(i, _):
            r0 = sid * NROW + i * EP
            pltpu.sync_copy(acc_sh.at[pl.ds(r0, EP)], ebuf)

            def nr(r, _):
                inv = 1.0 / (ebuf[r, HALF16, :] + 1e-16)
                for c in range(HALF16):
                    c2 = cid * HALF16 + c
                    v = ebuf[r, c, :] * inv + bv[c2, :]
                    if relu:
                        v = jnp.maximum(v, 0.0)
                    ebuf[r, c, :] = v
                return 0
            lax.fori_loop(0, EP, nr, 0)
            pltpu.sync_copy(ebuf, out_h.at[cid, pl.ds(r0, EP)])
            return 0
        lax.fori_loop(0, NROW // EP, ep, 0)

    return k


def _pad_rows(x):
    return jnp.pad(x, ((0, NPAD - N), (0, 0)))


def _gat_conv(xl, xr, src_p, dst_p, att, b, relu):
    C = xl.shape[1]
    C16 = C // 16
    HALF16 = C16 // 2
    G, GROUP, EP = (24, 28, 32) if C == 256 else (96, 28, 64)
    src3 = src_p.reshape(16, SLICE // G, G)
    dst3 = dst_p.reshape(16, SLICE // G, G)
    xl3 = _pad_rows(xl).reshape(NPAD, C16, 16)
    xr3 = _pad_rows(xr).reshape(NPAD, C16, 16)
    out = _gat_sc_kernel(C16, HALF16, G, GROUP, EP, True, relu)(
        src3, dst3, xl3, xr3, att.reshape(C16, 16), b.reshape(C16, 16))
    halves = out[:, :, :HALF16, :].reshape(2, NPAD, C // 2)
    return jnp.concatenate([halves[0], halves[1]], axis=-1)[:N]


def _gat_conv_pair(xlmu, xrmu, xlls, xrls, src_p, dst_p, attmu, bmu,
                   attls, bls):
    G = 128
    src3 = src_p.reshape(16, SLICE // G, G)
    dst3 = dst_p.reshape(16, SLICE // G, G)
    xl3 = _pad_rows(jnp.concatenate([xlmu, xlls], -1)).reshape(NPAD, 2, 16)
    xr3 = _pad_rows(jnp.concatenate([xrmu, xrls], -1)).reshape(NPAD, 2, 16)
    att2 = jnp.stack([attmu, attls])
    b2 = jnp.stack([bmu, bls])
    out = _gat_sc_kernel(2, 1, G, 84, 128, False, False)(
        src3, dst3, xl3, xr3, att2, b2)
    mu = out[0, :, 0, :].reshape(NPAD, 16)[:N]
    ls = out[1, :, 0, :].reshape(NPAD, 16)[:N]
    return mu, ls


def _matmul(x, w):
    M, K = x.shape
    Co = w.shape[1]
    BM = 1000

    def body(x_ref, w_ref, o_ref):
        o_ref[...] = jnp.dot(x_ref[...], w_ref[...],
                             preferred_element_type=jnp.float32)

    return pl.pallas_call(
        body,
        grid=(M // BM,),
        in_specs=[pl.BlockSpec((BM, K), lambda i: (i, 0)),
                  pl.BlockSpec((K, Co), lambda i: (0, 0))],
        out_specs=pl.BlockSpec((BM, Co), lambda i: (i, 0)),
        out_shape=jax.ShapeDtypeStruct((M, Co), jnp.float32),
    )(x, w)


def _leaky(x):
    return jnp.maximum(x, 0.0) + 0.2 * jnp.minimum(x, 0.0)


def _decoder(mu, logstd, eps, p):
    def body(mu_r, ls_r, eps_r, w1, b1, g1, be1, w2, b2, g2, be2, z_r, d_r):
        z = eps_r[...] * jnp.exp(ls_r[...]) + mu_r[...]
        z_r[...] = z
        a = jnp.dot(z, w1[...], preferred_element_type=jnp.float32) + b1[...]
        m = jnp.mean(a, axis=0, keepdims=True)
        v = jnp.mean((a - m) ** 2, axis=0, keepdims=True)
        a = _leaky(g1[...] * (a - m) / jnp.sqrt(v + 1e-5) + be1[...])
        d = jnp.dot(a, w2[...], preferred_element_type=jnp.float32) + b2[...]
        m2 = jnp.mean(d, axis=0, keepdims=True)
        v2 = jnp.mean((d - m2) ** 2, axis=0, keepdims=True)
        d_r[...] = _leaky(g2[...] * (d - m2) / jnp.sqrt(v2 + 1e-5) + be2[...])

    args = (mu, logstd, eps,
            p["W_d1"], p["b_d1"].reshape(1, -1),
            p["g1"].reshape(1, -1), p["be1"].reshape(1, -1),
            p["W_d2"], p["b_d2"].reshape(1, -1),
            p["g2"].reshape(1, -1), p["be2"].reshape(1, -1))
    return pl.pallas_call(
        body,
        out_shape=(jax.ShapeDtypeStruct((N, 16), jnp.float32),
                   jax.ShapeDtypeStruct((N, 256), jnp.float32)),
    )(*args)


def _adj(d):
    B = 400

    def body(a_ref, b_ref, o_ref):
        o_ref[...] = jax.nn.sigmoid(
            lax.dot_general(a_ref[...], b_ref[...],
                            (((1,), (1,)), ((), ())),
                            preferred_element_type=jnp.float32))

    return pl.pallas_call(
        body,
        grid=(N // B,),
        in_specs=[pl.BlockSpec((B, 256), lambda i: (i, 0)),
                  pl.BlockSpec((N, 256), lambda i: (0, 0))],
        out_specs=pl.BlockSpec((B, N), lambda i: (i, 0)),
        out_shape=jax.ShapeDtypeStruct((N, N), jnp.float32),
    )(d, d)


def _heads(d, p):
    BM = 1000
    OUT = p["W21"].shape[1]

    def body(d_ref, w1, c1, w2, c2, w3, c3, r1, r2, r3):
        x = d_ref[...]
        r1[...] = jnp.exp(
            jnp.dot(x, w1[...], preferred_element_type=jnp.float32) + c1[...])
        s = jnp.dot(x, w2[...], preferred_element_type=jnp.float32) + c2[...]
        r2[...] = jnp.clip(jax.nn.softplus(s), 1e-4, 1e4)
        r3[...] = jax.nn.sigmoid(
            jnp.dot(x, w3[...], preferred_element_type=jnp.float32) + c3[...])

    wspec = pl.BlockSpec((256, OUT), lambda i: (0, 0))
    bspec = pl.BlockSpec((1, OUT), lambda i: (0, 0))
    ospec = pl.BlockSpec((BM, OUT), lambda i: (i, 0))
    oshape = jax.ShapeDtypeStruct((N, OUT), jnp.float32)
    return pl.pallas_call(
        body,
        grid=(N // BM,),
        in_specs=[pl.BlockSpec((BM, 256), lambda i: (i, 0)),
                  wspec, bspec, wspec, bspec, wspec, bspec],
        out_specs=(ospec, ospec, ospec),
        out_shape=(oshape, oshape, oshape),
    )(d, p["W21"], p["b21"].reshape(1, -1),
      p["W22"], p["b22"].reshape(1, -1),
      p["W23"], p["b23"].reshape(1, -1))


def kernel(x, edge_index, spatial_coord, params):
    del spatial_coord
    loop = jnp.arange(N, dtype=edge_index.dtype)
    src_p = jnp.concatenate([
        edge_index[0], loop,
        jnp.zeros((ET_P - ET,), edge_index.dtype)])
    dst_p = jnp.concatenate([
        edge_index[1], loop,
        jnp.full((ET_P - ET,), NPAD - 1, edge_index.dtype)])

    def conv(h, name, relu):
        cp = params[name]
        C = cp["Wl"].shape[1]
        xlxr = _matmul(h, jnp.concatenate([cp["Wl"], cp["Wr"]], axis=1))
        return _gat_conv(xlxr[:, :C], xlxr[:, C:], src_p, dst_p,
                         cp["att"], cp["b"], relu)

    h = conv(x, "conv1", True)
    h = conv(h, "conv2", True)

    pmu, pls = params["conv_mu"], params["conv_logstd"]
    w4 = jnp.concatenate([pmu["Wl"], pmu["Wr"], pls["Wl"], pls["Wr"]], 1)
    xx = _matmul(h, w4)
    mu, logstd = _gat_conv_pair(
        xx[:, 0:16], xx[:, 16:32], xx[:, 32:48], xx[:, 48:64],
        src_p, dst_p, pmu["att"], pmu["b"], pls["att"], pls["b"])

    eps = jax.random.normal(jax.random.key(7), (N, 16), dtype=jnp.float32)
    z, d = _decoder(mu, logstd, eps, params)
    adj_pred = _adj(d)
    r1, r2, r3 = _heads(d, params)
    return (z, r1, r2, r3, mu, logstd, adj_pred)
